# baseline (speedup 1.0000x reference)
import functools

import jax
import jax.numpy as jnp
from jax import lax
from jax.experimental import pallas as pl
from jax.experimental.pallas import tpu as pltpu
from jax.experimental.pallas import tpu_sc as plsc

_N = 10000
_E = 320000
_F = 128
_NG = 2
_NS = 16
_C = 128
_NBUF = 2
_SEG = 32
_EPT_RAW = _E // _NS
_NCHUNK = -(-_EPT_RAW // (_SEG * _C)) * _SEG
_NSEG = _NCHUNK // _SEG
_EPT = _NCHUNK * _C
_R = _N + 112
_ZR = _R // _NS
_DW = 16


def _agg_body(src_h, dst_h, u_h, zeros_h, out_h, idx_src, idx_dst, buf, acc,
              sem):
    c = lax.axis_index("c")
    s = lax.axis_index("s")
    pltpu.sync_copy(zeros_h, acc.at[pl.ds(s * _ZR, _ZR)])
    plsc.subcore_barrier()
    row = (c * _NS + s) * _NCHUNK

    def seg_body(g, carry):
        base = row + g * _SEG
        pltpu.sync_copy(src_h.at[pl.ds(base, _SEG)], idx_src)
        pltpu.sync_copy(dst_h.at[pl.ds(base, _SEG)], idx_dst)

        for b in range(_NBUF):
            pltpu.async_copy(u_h.at[idx_src.at[b]], buf.at[b], sem)

        def step(i, carry2):
            j = i * _NBUF
            for b in range(_NBUF):
                jj = j + b
                pltpu.make_async_copy(u_h.at[idx_src.at[jj]], buf.at[b],
                                      sem).wait()
                pltpu.sync_copy(buf.at[b], acc.at[idx_dst.at[jj]], add=True)
                pltpu.async_copy(u_h.at[idx_src.at[jj + _NBUF]], buf.at[b],
                                 sem)
            return carry2

        lax.fori_loop(0, _SEG // _NBUF - 1, step, 0)
        tail = _SEG - _NBUF
        for b in range(_NBUF):
            pltpu.make_async_copy(u_h.at[idx_src.at[tail + b]], buf.at[b],
                                  sem).wait()
            pltpu.sync_copy(buf.at[b], acc.at[idx_dst.at[tail + b]], add=True)
        return carry

    lax.fori_loop(0, _NSEG, seg_body, 0)
    plsc.subcore_barrier()
    pltpu.sync_copy(acc.at[pl.ds(s * _ZR, _ZR)],
                    out_h.at[pl.ds(c * _R + s * _ZR, _ZR)])


@functools.cache
def _sc_kernels():
    mesh = plsc.VectorSubcoreMesh(
        core_axis_name="c", subcore_axis_name="s",
        num_cores=_NG, num_subcores=_NS)
    agg = pl.kernel(
        _agg_body,
        jax.ShapeDtypeStruct((_NG * _R, _F), jnp.float32),
        mesh=mesh,
        scratch_types=[
            pltpu.VMEM((_SEG, _C), jnp.int32),
            pltpu.VMEM((_SEG, _C), jnp.int32),
            pltpu.VMEM((_NBUF, _C, _F), jnp.float32),
            pltpu.VMEM_SHARED((_R, _F), jnp.float32),
            pltpu.SemaphoreType.DMA,
        ],
    )
    return agg


_B = 1000
_NB = _N // _B


def _l1(x):
    return x / jnp.maximum(jnp.sum(jnp.abs(x), axis=1, keepdims=True), 1e-12)


def _k1_body(rwr_ref, x_ref, deg_ref, linW_ref, linb_ref, W1_ref,
             pos_ref, u_ref):
    dinv = lax.rsqrt(deg_ref[0, :, 0:1] + 1.0)
    p = jnp.dot(rwr_ref[0], linW_ref[...],
                preferred_element_type=jnp.float32) + linb_ref[...]
    pos_ref[0] = _l1(p)
    u_ref[0] = dinv * jnp.dot(x_ref[0], W1_ref[...],
                              preferred_element_type=jnp.float32)


def _k2_body(agg_ref, u_ref, deg_ref, W2_ref, b_ref, conv_ref, unext_ref):
    dinv = lax.rsqrt(deg_ref[0, :, 0:1] + 1.0)
    conv = dinv * (agg_ref[0] + u_ref[0]) + b_ref[...]
    conv_ref[0] = conv
    unext_ref[0] = dinv * jnp.dot(conv, W2_ref[...],
                                  preferred_element_type=jnp.float32)


def _k3_body(conv2_ref, pos_ref, Wt_ref, Wb_ref, cb_ref, out_ref):
    gcn = _l1(conv2_ref[0])
    z = (jnp.dot(pos_ref[0], Wt_ref[...], preferred_element_type=jnp.float32)
         + jnp.dot(gcn, Wb_ref[...], preferred_element_type=jnp.float32)
         + cb_ref[...])
    out_ref[0] = _l1(z)


def _row_spec():
    return pl.BlockSpec((1, _B, _F), lambda g, i: (g, i, 0))


def _deg_spec():
    return pl.BlockSpec((1, _B, _F), lambda g, i: (g, i, 0))


def _w_spec():
    return pl.BlockSpec((_F, _F), lambda g, i: (0, 0))


def _b_spec():
    return pl.BlockSpec((1, _F), lambda g, i: (0, 0))


_k1 = pl.pallas_call(
    _k1_body,
    grid=(_NG, _NB),
    in_specs=[_row_spec(), _row_spec(), _deg_spec(), _w_spec(), _b_spec(),
              _w_spec()],
    out_specs=[_row_spec(), _row_spec()],
    out_shape=[jax.ShapeDtypeStruct((_NG, _N, _F), jnp.float32),
               jax.ShapeDtypeStruct((_NG, _N, _F), jnp.float32)],
)

_k2 = pl.pallas_call(
    _k2_body,
    grid=(_NG, _NB),
    in_specs=[_row_spec(), _row_spec(), _deg_spec(), _w_spec(), _b_spec()],
    out_specs=[_row_spec(), _row_spec()],
    out_shape=[jax.ShapeDtypeStruct((_NG, _N, _F), jnp.float32),
               jax.ShapeDtypeStruct((_NG, _N, _F), jnp.float32)],
)

_k3 = pl.pallas_call(
    _k3_body,
    grid=(_NG, _NB),
    in_specs=[_row_spec(), _row_spec(), _w_spec(), _w_spec(), _b_spec()],
    out_specs=_row_spec(),
    out_shape=jax.ShapeDtypeStruct((_NG, _N, _F), jnp.float32),
)


def _prep_edges(ei, g):
    pad = _EPT - _EPT_RAW
    src = ei[0].reshape(_NS, _EPT_RAW)
    dst = ei[1].reshape(_NS, _EPT_RAW)
    src = jnp.pad(src, ((0, 0), (0, pad))) + g * _N
    dst = jnp.pad(dst, ((0, 0), (0, pad)), constant_values=_N)
    return (src.reshape(_NS * _NCHUNK, _C), dst.reshape(_NS * _NCHUNK, _C))


def kernel(rwr1_emd, rwr2_emd, x1, edge_index1, x2, edge_index2,
           lin_W, lin_b, gcn_W1, gcn_b1, gcn_W2, gcn_b2, comb_W, comb_b):
    s1, d1 = _prep_edges(edge_index1, 0)
    s2, d2 = _prep_edges(edge_index2, 1)
    src_all = jnp.concatenate([s1, s2], axis=0)
    dst_all = jnp.concatenate([d1, d2], axis=0)

    zeros_f = jnp.zeros((_ZR, _F), jnp.float32)

    _agg = _sc_kernels()
    src_zero = jnp.zeros_like(src_all)
    ones_u = jnp.ones((16, _F), jnp.float32)
    deg = _agg(src_zero, dst_all, ones_u, zeros_f)
    deg3 = deg.reshape(_NG, _R, _F)

    rwr = jnp.stack([rwr1_emd, rwr2_emd])
    x = jnp.stack([x1, x2])
    pos_n, u = _k1(rwr, x, deg3, lin_W, lin_b.reshape(1, _F), gcn_W1)

    def layer(k, carry):
        u_cur, _ = carry
        a = _agg(src_all, dst_all, u_cur.reshape(_NG * _N, _F), zeros_f)
        a = a.reshape(_NG, _R, _F)
        b_k = jnp.where(k == 0, gcn_b1, gcn_b2).reshape(1, _F)
        conv, unext = _k2(a, u_cur, deg3, gcn_W2, b_k)
        return (unext, conv)

    conv0 = jnp.zeros((_NG, _N, _F), jnp.float32)
    two = 2 + jnp.minimum(edge_index1[0, 0], 0) * 0
    _, conv2 = lax.fori_loop(0, two, layer, (u, conv0))

    emd = _k3(conv2, pos_n, comb_W[:_F], comb_W[_F:], comb_b.reshape(1, _F))
    return (emd[0], emd[1])


# speedup vs baseline: 15.6314x; 15.6314x over previous
import functools

import jax
import jax.numpy as jnp
from jax import lax
from jax.experimental import pallas as pl
from jax.experimental.pallas import tpu as pltpu
from jax.experimental.pallas import tpu_sc as plsc

_N = 10000
_E = 320000
_F = 128
_NG = 2
_NS = 16
_C = 128
_NBUF = 2
_SEG = 32
_EPT_RAW = _E // _NS
_NCHUNK = -(-_EPT_RAW // (_SEG * _C)) * _SEG
_NSEG = _NCHUNK // _SEG
_EPT = _NCHUNK * _C
_R = _N + 112
_ZR = _R // _NS
_DW = 16


def _agg_body(src_h, dst_h, u_h, zeros_h, out_h, idx_src, idx_dst, buf, acc,
              sem):
    c = lax.axis_index("c")
    s = lax.axis_index("s")
    pltpu.sync_copy(zeros_h, acc.at[pl.ds(s * _ZR, _ZR)])
    plsc.subcore_barrier()
    row = (c * _NS + s) * _NCHUNK

    def seg_body(g, carry):
        base = row + g * _SEG
        pltpu.sync_copy(src_h.at[pl.ds(base, _SEG)], idx_src)
        pltpu.sync_copy(dst_h.at[pl.ds(base, _SEG)], idx_dst)

        for b in range(_NBUF):
            pltpu.async_copy(u_h.at[idx_src.at[b]], buf.at[b], sem)

        def step(i, carry2):
            j = i * _NBUF
            for b in range(_NBUF):
                jj = j + b
                pltpu.make_async_copy(u_h.at[idx_src.at[jj]], buf.at[b],
                                      sem).wait()
                pltpu.sync_copy(buf.at[b], acc.at[idx_dst.at[jj]], add=True)
                pltpu.async_copy(u_h.at[idx_src.at[jj + _NBUF]], buf.at[b],
                                 sem)
            return carry2

        lax.fori_loop(0, _SEG // _NBUF - 1, step, 0)
        tail = _SEG - _NBUF
        for b in range(_NBUF):
            pltpu.make_async_copy(u_h.at[idx_src.at[tail + b]], buf.at[b],
                                  sem).wait()
            pltpu.sync_copy(buf.at[b], acc.at[idx_dst.at[tail + b]], add=True)
        return carry

    lax.fori_loop(0, _NSEG, seg_body, 0)
    plsc.subcore_barrier()
    pltpu.sync_copy(acc.at[pl.ds(s * _ZR, _ZR)],
                    out_h.at[pl.ds(c * _R + s * _ZR, _ZR)])


@functools.cache
def _sc_kernels():
    mesh = plsc.VectorSubcoreMesh(
        core_axis_name="c", subcore_axis_name="s",
        num_cores=_NG, num_subcores=_NS)
    agg = pl.kernel(
        _agg_body,
        jax.ShapeDtypeStruct((_NG * _R, _F), jnp.float32),
        mesh=mesh,
        scratch_types=[
            pltpu.VMEM((_SEG, _C), jnp.int32),
            pltpu.VMEM((_SEG, _C), jnp.int32),
            pltpu.VMEM((_NBUF, _C, _F), jnp.float32),
            pltpu.VMEM_SHARED((_R, _F), jnp.float32),
            pltpu.SemaphoreType.DMA,
        ],
    )
    return agg


_B = 1000
_NB = _N // _B


def _l1(x):
    return x / jnp.maximum(jnp.sum(jnp.abs(x), axis=1, keepdims=True), 1e-12)


def _k1_body(rwr_ref, x_ref, deg_ref, linW_ref, linb_ref, W1_ref,
             pos_ref, u_ref):
    dinv = lax.rsqrt(deg_ref[0, :, 0:1] + 1.0)
    p = jnp.dot(rwr_ref[0], linW_ref[...],
                preferred_element_type=jnp.float32) + linb_ref[...]
    pos_ref[0] = _l1(p)
    u_ref[0] = dinv * jnp.dot(x_ref[0], W1_ref[...],
                              preferred_element_type=jnp.float32)


def _k2_body(agg_ref, u_ref, deg_ref, W2_ref, b_ref, conv_ref, unext_ref):
    dinv = lax.rsqrt(deg_ref[0, :, 0:1] + 1.0)
    conv = dinv * (agg_ref[0] + u_ref[0]) + b_ref[...]
    conv_ref[0] = conv
    unext_ref[0] = dinv * jnp.dot(conv, W2_ref[...],
                                  preferred_element_type=jnp.float32)


def _k3_body(conv2_ref, pos_ref, Wt_ref, Wb_ref, cb_ref, out_ref):
    gcn = _l1(conv2_ref[0])
    z = (jnp.dot(pos_ref[0], Wt_ref[...], preferred_element_type=jnp.float32)
         + jnp.dot(gcn, Wb_ref[...], preferred_element_type=jnp.float32)
         + cb_ref[...])
    out_ref[0] = _l1(z)


def _row_spec():
    return pl.BlockSpec((1, _B, _F), lambda g, i: (g, i, 0))


def _deg_spec():
    return pl.BlockSpec((1, _B, _F), lambda g, i: (g, i, 0))


def _w_spec():
    return pl.BlockSpec((_F, _F), lambda g, i: (0, 0))


def _b_spec():
    return pl.BlockSpec((1, _F), lambda g, i: (0, 0))


_k1 = pl.pallas_call(
    _k1_body,
    grid=(_NG, _NB),
    in_specs=[_row_spec(), _row_spec(), _deg_spec(), _w_spec(), _b_spec(),
              _w_spec()],
    out_specs=[_row_spec(), _row_spec()],
    out_shape=[jax.ShapeDtypeStruct((_NG, _N, _F), jnp.float32),
               jax.ShapeDtypeStruct((_NG, _N, _F), jnp.float32)],
)

_k2 = pl.pallas_call(
    _k2_body,
    grid=(_NG, _NB),
    in_specs=[_row_spec(), _row_spec(), _deg_spec(), _w_spec(), _b_spec()],
    out_specs=[_row_spec(), _row_spec()],
    out_shape=[jax.ShapeDtypeStruct((_NG, _N, _F), jnp.float32),
               jax.ShapeDtypeStruct((_NG, _N, _F), jnp.float32)],
)

_k3 = pl.pallas_call(
    _k3_body,
    grid=(_NG, _NB),
    in_specs=[_row_spec(), _row_spec(), _w_spec(), _w_spec(), _b_spec()],
    out_specs=_row_spec(),
    out_shape=jax.ShapeDtypeStruct((_NG, _N, _F), jnp.float32),
)


def _prep_edges(ei, g):
    pad = _EPT - _EPT_RAW
    src = ei[0].reshape(_NS, _EPT_RAW)
    dst = ei[1].reshape(_NS, _EPT_RAW)
    src = jnp.pad(src, ((0, 0), (0, pad))) + g * _N
    dst = jnp.pad(dst, ((0, 0), (0, pad)), constant_values=_N)
    return (src.reshape(_NS * _NCHUNK, _C), dst.reshape(_NS * _NCHUNK, _C))


def kernel(rwr1_emd, rwr2_emd, x1, edge_index1, x2, edge_index2,
           lin_W, lin_b, gcn_W1, gcn_b1, gcn_W2, gcn_b2, comb_W, comb_b):
    s1, d1 = _prep_edges(edge_index1, 0)
    s2, d2 = _prep_edges(edge_index2, 1)
    src_all = jnp.concatenate([s1, s2], axis=0)
    dst_all = jnp.concatenate([d1, d2], axis=0)

    zeros_f = jnp.zeros((_ZR, _F), jnp.float32)

    _agg = _sc_kernels()
    ones_u = jnp.ones((_NG * _N, _F), jnp.float32)
    deg = _agg(src_all, dst_all, ones_u, zeros_f)
    deg3 = deg.reshape(_NG, _R, _F)

    rwr = jnp.stack([rwr1_emd, rwr2_emd])
    x = jnp.stack([x1, x2])
    pos_n, u = _k1(rwr, x, deg3, lin_W, lin_b.reshape(1, _F), gcn_W1)

    def layer(k, carry):
        u_cur, _ = carry
        a = _agg(src_all, dst_all, u_cur.reshape(_NG * _N, _F), zeros_f)
        a = a.reshape(_NG, _R, _F)
        b_k = jnp.where(k == 0, gcn_b1, gcn_b2).reshape(1, _F)
        conv, unext = _k2(a, u_cur, deg3, gcn_W2, b_k)
        return (unext, conv)

    conv0 = jnp.zeros((_NG, _N, _F), jnp.float32)
    two = 2 + jnp.minimum(edge_index1[0, 0], 0) * 0
    _, conv2 = lax.fori_loop(0, two, layer, (u, conv0))

    emd = _k3(conv2, pos_n, comb_W[:_F], comb_W[_F:], comb_b.reshape(1, _F))
    return (emd[0], emd[1])


# speedup vs baseline: 20.2327x; 1.2944x over previous
import functools

import jax
import jax.numpy as jnp
from jax import lax
from jax.experimental import pallas as pl
from jax.experimental.pallas import tpu as pltpu
from jax.experimental.pallas import tpu_sc as plsc

_N = 10000
_E = 320000
_F = 128
_NG = 2
_NS = 16
_C = 128
_NBUF = 2
_SEG = 32
_EPT_RAW = _E // _NS
_NCHUNK = -(-_EPT_RAW // (_SEG * _C)) * _SEG
_NSEG = _NCHUNK // _SEG
_EPT = _NCHUNK * _C
_R = _N + 112
_ZR = _R // _NS
_DW = 16


def _deg_body(dst_h, erows_h, zeros_h, out_h, idx_dst, ebuf, acc):
    c = lax.axis_index("c")
    s = lax.axis_index("s")
    pltpu.sync_copy(erows_h, ebuf)
    pltpu.sync_copy(zeros_h, acc.at[pl.ds(s * _ZR, _ZR)])
    plsc.subcore_barrier()
    row = (c * _NS + s) * _NCHUNK

    def seg_body(g, carry):
        pltpu.sync_copy(dst_h.at[pl.ds(row + g * _SEG, _SEG)], idx_dst)

        def step(j, carry2):
            pltpu.sync_copy(ebuf, acc.at[idx_dst.at[j]], add=True)
            return carry2

        lax.fori_loop(0, _SEG, step, 0)
        return carry

    lax.fori_loop(0, _NSEG, seg_body, 0)
    plsc.subcore_barrier()
    pltpu.sync_copy(acc.at[pl.ds(s * _ZR, _ZR)],
                    out_h.at[pl.ds(c * _R + s * _ZR, _ZR)])


def _agg_body(src_h, dst_h, u_h, zeros_h, out_h, idx_src, idx_dst, buf, acc,
              sem):
    c = lax.axis_index("c")
    s = lax.axis_index("s")
    pltpu.sync_copy(zeros_h, acc.at[pl.ds(s * _ZR, _ZR)])
    plsc.subcore_barrier()
    row = (c * _NS + s) * _NCHUNK

    def seg_body(g, carry):
        base = row + g * _SEG
        pltpu.sync_copy(src_h.at[pl.ds(base, _SEG)], idx_src)
        pltpu.sync_copy(dst_h.at[pl.ds(base, _SEG)], idx_dst)

        for b in range(_NBUF):
            pltpu.async_copy(u_h.at[idx_src.at[b]], buf.at[b], sem)

        def step(i, carry2):
            j = i * _NBUF
            for b in range(_NBUF):
                jj = j + b
                pltpu.make_async_copy(u_h.at[idx_src.at[jj]], buf.at[b],
                                      sem).wait()
                pltpu.sync_copy(buf.at[b], acc.at[idx_dst.at[jj]], add=True)
                pltpu.async_copy(u_h.at[idx_src.at[jj + _NBUF]], buf.at[b],
                                 sem)
            return carry2

        lax.fori_loop(0, _SEG // _NBUF - 1, step, 0)
        tail = _SEG - _NBUF
        for b in range(_NBUF):
            pltpu.make_async_copy(u_h.at[idx_src.at[tail + b]], buf.at[b],
                                  sem).wait()
            pltpu.sync_copy(buf.at[b], acc.at[idx_dst.at[tail + b]], add=True)
        return carry

    lax.fori_loop(0, _NSEG, seg_body, 0)
    plsc.subcore_barrier()
    pltpu.sync_copy(acc.at[pl.ds(s * _ZR, _ZR)],
                    out_h.at[pl.ds(c * _R + s * _ZR, _ZR)])


@functools.cache
def _sc_kernels():
    mesh = plsc.VectorSubcoreMesh(
        core_axis_name="c", subcore_axis_name="s",
        num_cores=_NG, num_subcores=_NS)
    deg = pl.kernel(
        _deg_body,
        jax.ShapeDtypeStruct((_NG * _R, _F), jnp.float32),
        mesh=mesh,
        scratch_types=[
            pltpu.VMEM((_SEG, _C), jnp.int32),
            pltpu.VMEM((_C, _F), jnp.float32),
            pltpu.VMEM_SHARED((_R, _F), jnp.float32),
        ],
    )
    agg = pl.kernel(
        _agg_body,
        jax.ShapeDtypeStruct((_NG * _R, _F), jnp.float32),
        mesh=mesh,
        scratch_types=[
            pltpu.VMEM((_SEG, _C), jnp.int32),
            pltpu.VMEM((_SEG, _C), jnp.int32),
            pltpu.VMEM((_NBUF, _C, _F), jnp.float32),
            pltpu.VMEM_SHARED((_R, _F), jnp.float32),
            pltpu.SemaphoreType.DMA,
        ],
    )
    return deg, agg


_B = 1000
_NB = _N // _B


def _l1(x):
    return x / jnp.maximum(jnp.sum(jnp.abs(x), axis=1, keepdims=True), 1e-12)


def _k1_body(rwr_ref, x_ref, deg_ref, linW_ref, linb_ref, W1_ref,
             pos_ref, u_ref):
    dinv = lax.rsqrt(deg_ref[0, :, 0:1] + 1.0)
    p = jnp.dot(rwr_ref[0], linW_ref[...],
                preferred_element_type=jnp.float32) + linb_ref[...]
    pos_ref[0] = _l1(p)
    u_ref[0] = dinv * jnp.dot(x_ref[0], W1_ref[...],
                              preferred_element_type=jnp.float32)


def _k2_body(agg_ref, u_ref, deg_ref, W2_ref, b_ref, conv_ref, unext_ref):
    dinv = lax.rsqrt(deg_ref[0, :, 0:1] + 1.0)
    conv = dinv * (agg_ref[0] + u_ref[0]) + b_ref[...]
    conv_ref[0] = conv
    unext_ref[0] = dinv * jnp.dot(conv, W2_ref[...],
                                  preferred_element_type=jnp.float32)


def _k3_body(conv2_ref, pos_ref, Wt_ref, Wb_ref, cb_ref, out_ref):
    gcn = _l1(conv2_ref[0])
    z = (jnp.dot(pos_ref[0], Wt_ref[...], preferred_element_type=jnp.float32)
         + jnp.dot(gcn, Wb_ref[...], preferred_element_type=jnp.float32)
         + cb_ref[...])
    out_ref[0] = _l1(z)


def _row_spec():
    return pl.BlockSpec((1, _B, _F), lambda g, i: (g, i, 0))


def _deg_spec():
    return pl.BlockSpec((1, _B, _F), lambda g, i: (g, i, 0))


def _w_spec():
    return pl.BlockSpec((_F, _F), lambda g, i: (0, 0))


def _b_spec():
    return pl.BlockSpec((1, _F), lambda g, i: (0, 0))


_k1 = pl.pallas_call(
    _k1_body,
    grid=(_NG, _NB),
    in_specs=[_row_spec(), _row_spec(), _deg_spec(), _w_spec(), _b_spec(),
              _w_spec()],
    out_specs=[_row_spec(), _row_spec()],
    out_shape=[jax.ShapeDtypeStruct((_NG, _N, _F), jnp.float32),
               jax.ShapeDtypeStruct((_NG, _N, _F), jnp.float32)],
)

_k2 = pl.pallas_call(
    _k2_body,
    grid=(_NG, _NB),
    in_specs=[_row_spec(), _row_spec(), _deg_spec(), _w_spec(), _b_spec()],
    out_specs=[_row_spec(), _row_spec()],
    out_shape=[jax.ShapeDtypeStruct((_NG, _N, _F), jnp.float32),
               jax.ShapeDtypeStruct((_NG, _N, _F), jnp.float32)],
)

_k3 = pl.pallas_call(
    _k3_body,
    grid=(_NG, _NB),
    in_specs=[_row_spec(), _row_spec(), _w_spec(), _w_spec(), _b_spec()],
    out_specs=_row_spec(),
    out_shape=jax.ShapeDtypeStruct((_NG, _N, _F), jnp.float32),
)


def _prep_edges(ei, g):
    pad = _EPT - _EPT_RAW
    src = ei[0].reshape(_NS, _EPT_RAW)
    dst = ei[1].reshape(_NS, _EPT_RAW)
    src = jnp.pad(src, ((0, 0), (0, pad))) + g * _N
    dst = jnp.pad(dst, ((0, 0), (0, pad)), constant_values=_N)
    return (src.reshape(_NS * _NCHUNK, _C), dst.reshape(_NS * _NCHUNK, _C))


def kernel(rwr1_emd, rwr2_emd, x1, edge_index1, x2, edge_index2,
           lin_W, lin_b, gcn_W1, gcn_b1, gcn_W2, gcn_b2, comb_W, comb_b):
    s1, d1 = _prep_edges(edge_index1, 0)
    s2, d2 = _prep_edges(edge_index2, 1)
    src_all = jnp.concatenate([s1, s2], axis=0)
    dst_all = jnp.concatenate([d1, d2], axis=0)

    zeros_f = jnp.zeros((_ZR, _F), jnp.float32)
    ones_e = jnp.ones((_C, _F), jnp.float32)

    _deg, _agg = _sc_kernels()
    deg = _deg(dst_all, ones_e, zeros_f)
    deg3 = deg.reshape(_NG, _R, _F)

    rwr = jnp.stack([rwr1_emd, rwr2_emd])
    x = jnp.stack([x1, x2])
    pos_n, u = _k1(rwr, x, deg3, lin_W, lin_b.reshape(1, _F), gcn_W1)

    def layer(k, carry):
        u_cur, _ = carry
        a = _agg(src_all, dst_all, u_cur.reshape(_NG * _N, _F), zeros_f)
        a = a.reshape(_NG, _R, _F)
        b_k = jnp.where(k == 0, gcn_b1, gcn_b2).reshape(1, _F)
        conv, unext = _k2(a, u_cur, deg3, gcn_W2, b_k)
        return (unext, conv)

    conv0 = jnp.zeros((_NG, _N, _F), jnp.float32)
    two = 2 + jnp.minimum(edge_index1[0, 0], 0) * 0
    _, conv2 = lax.fori_loop(0, two, layer, (u, conv0))

    emd = _k3(conv2, pos_n, comb_W[:_F], comb_W[_F:], comb_b.reshape(1, _F))
    return (emd[0], emd[1])


# speedup vs baseline: 20.2503x; 1.0009x over previous
import functools

import jax
import jax.numpy as jnp
from jax import lax
from jax.experimental import pallas as pl
from jax.experimental.pallas import tpu as pltpu
from jax.experimental.pallas import tpu_sc as plsc

_N = 10000
_E = 320000
_F = 128
_NG = 2
_NS = 16
_C = 128
_NBUF = 2
_SEG = 32
_EPT_RAW = _E // _NS
_NCHUNK = -(-_EPT_RAW // (_SEG * _C)) * _SEG
_NSEG = _NCHUNK // _SEG
_EPT = _NCHUNK * _C
_R = _N + 112
_ZR = _R // _NS


def _deg_body(dst_h, erows_h, zeros_h, out_h, idx_dst, ebuf, acc):
    c = lax.axis_index("c")
    s = lax.axis_index("s")
    pltpu.sync_copy(erows_h, ebuf)
    pltpu.sync_copy(zeros_h, acc.at[pl.ds(s * _ZR, _ZR)])
    plsc.subcore_barrier()
    row = (c * _NS + s) * _NCHUNK

    def seg_body(g, carry):
        pltpu.sync_copy(dst_h.at[pl.ds(row + g * _SEG, _SEG)], idx_dst)

        def step(j, carry2):
            pltpu.sync_copy(ebuf, acc.at[idx_dst.at[j]], add=True)
            return carry2

        lax.fori_loop(0, _SEG, step, 0)
        return carry

    lax.fori_loop(0, _NSEG, seg_body, 0)
    plsc.subcore_barrier()
    pltpu.sync_copy(acc.at[pl.ds(s * _ZR, _ZR)],
                    out_h.at[pl.ds(c * _R + s * _ZR, _ZR)])


def _agg_body(src_h, dst_h, u_h, zeros_h, out_h, idx_src, idx_dst, buf, acc,
              sem):
    c = lax.axis_index("c")
    s = lax.axis_index("s")
    pltpu.sync_copy(zeros_h, acc.at[pl.ds(s * _ZR, _ZR)])
    plsc.subcore_barrier()
    row = (c * _NS + s) * _NCHUNK

    def seg_body(g, carry):
        base = row + g * _SEG
        pltpu.sync_copy(src_h.at[pl.ds(base, _SEG)], idx_src)
        pltpu.sync_copy(dst_h.at[pl.ds(base, _SEG)], idx_dst)

        for b in range(_NBUF):
            pltpu.async_copy(u_h.at[idx_src.at[b]], buf.at[b], sem)

        def step(i, carry2):
            j = i * _NBUF
            for b in range(_NBUF):
                jj = j + b
                pltpu.make_async_copy(u_h.at[idx_src.at[jj]], buf.at[b],
                                      sem).wait()
                pltpu.sync_copy(buf.at[b], acc.at[idx_dst.at[jj]], add=True)
                pltpu.async_copy(u_h.at[idx_src.at[jj + _NBUF]], buf.at[b],
                                 sem)
            return carry2

        lax.fori_loop(0, _SEG // _NBUF - 1, step, 0)
        tail = _SEG - _NBUF
        for b in range(_NBUF):
            pltpu.make_async_copy(u_h.at[idx_src.at[tail + b]], buf.at[b],
                                  sem).wait()
            pltpu.sync_copy(buf.at[b], acc.at[idx_dst.at[tail + b]], add=True)
        return carry

    lax.fori_loop(0, _NSEG, seg_body, 0)
    plsc.subcore_barrier()
    pltpu.sync_copy(acc.at[pl.ds(s * _ZR, _ZR)],
                    out_h.at[pl.ds(c * _R + s * _ZR, _ZR)])


@functools.cache
def _sc_kernels():
    mesh = plsc.VectorSubcoreMesh(
        core_axis_name="c", subcore_axis_name="s",
        num_cores=_NG, num_subcores=_NS)
    deg = pl.kernel(
        _deg_body,
        jax.ShapeDtypeStruct((_NG * _R, _F), jnp.float32),
        mesh=mesh,
        scratch_types=[
            pltpu.VMEM((_SEG, _C), jnp.int32),
            pltpu.VMEM((_C, _F), jnp.float32),
            pltpu.VMEM_SHARED((_R, _F), jnp.float32),
        ],
    )
    agg = pl.kernel(
        _agg_body,
        jax.ShapeDtypeStruct((_NG * _R, _F), jnp.float32),
        mesh=mesh,
        scratch_types=[
            pltpu.VMEM((_SEG, _C), jnp.int32),
            pltpu.VMEM((_SEG, _C), jnp.int32),
            pltpu.VMEM((_NBUF, _C, _F), jnp.float32),
            pltpu.VMEM_SHARED((_R, _F), jnp.float32),
            pltpu.SemaphoreType.DMA,
        ],
    )
    return deg, agg


_B = 1000
_NB = _N // _B


def _l1(x):
    return x / jnp.maximum(jnp.sum(jnp.abs(x), axis=1, keepdims=True), 1e-12)


def _k1_body(rwr_ref, x_ref, deg_ref, linW_ref, linb_ref, W1_ref,
             pos_ref, u_ref):
    dinv = lax.rsqrt(deg_ref[0, :, 0:1] + 1.0)
    p = jnp.dot(rwr_ref[0], linW_ref[...],
                preferred_element_type=jnp.float32) + linb_ref[...]
    pos_ref[0] = _l1(p)
    u_ref[0] = dinv * jnp.dot(x_ref[0], W1_ref[...],
                              preferred_element_type=jnp.float32)


def _k2_body(agg_ref, u_ref, deg_ref, W2_ref, b_ref, conv_ref, unext_ref):
    dinv = lax.rsqrt(deg_ref[0, :, 0:1] + 1.0)
    conv = dinv * (agg_ref[0] + u_ref[0]) + b_ref[...]
    conv_ref[0] = conv
    unext_ref[0] = dinv * jnp.dot(conv, W2_ref[...],
                                  preferred_element_type=jnp.float32)


def _k3_body(conv2_ref, pos_ref, Wt_ref, Wb_ref, cb_ref, out_ref):
    gcn = _l1(conv2_ref[0])
    z = (jnp.dot(pos_ref[0], Wt_ref[...], preferred_element_type=jnp.float32)
         + jnp.dot(gcn, Wb_ref[...], preferred_element_type=jnp.float32)
         + cb_ref[...])
    out_ref[0] = _l1(z)


def _row_spec():
    return pl.BlockSpec((1, _B, _F), lambda g, i: (g, i, 0))


def _deg_spec():
    return pl.BlockSpec((1, _B, _F), lambda g, i: (g, i, 0))


def _w_spec():
    return pl.BlockSpec((_F, _F), lambda g, i: (0, 0))


def _b_spec():
    return pl.BlockSpec((1, _F), lambda g, i: (0, 0))


_k1 = pl.pallas_call(
    _k1_body,
    grid=(_NG, _NB),
    in_specs=[_row_spec(), _row_spec(), _deg_spec(), _w_spec(), _b_spec(),
              _w_spec()],
    out_specs=[_row_spec(), _row_spec()],
    out_shape=[jax.ShapeDtypeStruct((_NG, _N, _F), jnp.float32),
               jax.ShapeDtypeStruct((_NG, _N, _F), jnp.float32)],
)

_k2 = pl.pallas_call(
    _k2_body,
    grid=(_NG, _NB),
    in_specs=[_row_spec(), _row_spec(), _deg_spec(), _w_spec(), _b_spec()],
    out_specs=[_row_spec(), _row_spec()],
    out_shape=[jax.ShapeDtypeStruct((_NG, _N, _F), jnp.float32),
               jax.ShapeDtypeStruct((_NG, _N, _F), jnp.float32)],
)

_k3 = pl.pallas_call(
    _k3_body,
    grid=(_NG, _NB),
    in_specs=[_row_spec(), _row_spec(), _w_spec(), _w_spec(), _b_spec()],
    out_specs=_row_spec(),
    out_shape=jax.ShapeDtypeStruct((_NG, _N, _F), jnp.float32),
)


def _prep_edges(ei, g):
    pad = _EPT - _EPT_RAW
    src = ei[0].reshape(_NS, _EPT_RAW)
    dst = ei[1].reshape(_NS, _EPT_RAW)
    src = jnp.pad(src, ((0, 0), (0, pad))) + g * _N
    dst = jnp.pad(dst, ((0, 0), (0, pad)), constant_values=_N)
    return (src.reshape(_NS * _NCHUNK, _C), dst.reshape(_NS * _NCHUNK, _C))


def kernel(rwr1_emd, rwr2_emd, x1, edge_index1, x2, edge_index2,
           lin_W, lin_b, gcn_W1, gcn_b1, gcn_W2, gcn_b2, comb_W, comb_b):
    s1, d1 = _prep_edges(edge_index1, 0)
    s2, d2 = _prep_edges(edge_index2, 1)
    src_all = jnp.concatenate([s1, s2], axis=0)
    dst_all = jnp.concatenate([d1, d2], axis=0)

    zeros_f = jnp.zeros((_ZR, _F), jnp.float32)
    ones_e = jnp.ones((_C, _F), jnp.float32)

    _deg, _agg = _sc_kernels()
    deg = _deg(dst_all, ones_e, zeros_f)
    deg3 = deg.reshape(_NG, _R, _F)

    rwr = jnp.stack([rwr1_emd, rwr2_emd])
    x = jnp.stack([x1, x2])
    pos_n, u = _k1(rwr, x, deg3, lin_W, lin_b.reshape(1, _F), gcn_W1)

    def layer(k, carry):
        u_cur, _ = carry
        a = _agg(src_all, dst_all, u_cur.reshape(_NG * _N, _F), zeros_f)
        a = a.reshape(_NG, _R, _F)
        b_k = jnp.where(k == 0, gcn_b1, gcn_b2).reshape(1, _F)
        conv, unext = _k2(a, u_cur, deg3, gcn_W2, b_k)
        return (unext, conv)

    conv0 = jnp.zeros((_NG, _N, _F), jnp.float32)
    two = 2 + jnp.minimum(edge_index1[0, 0], 0) * 0
    _, conv2 = lax.fori_loop(0, two, layer, (u, conv0))

    emd = _k3(conv2, pos_n, comb_W[:_F], comb_W[_F:], comb_b.reshape(1, _F))
    return (emd[0], emd[1])
